# baseline (speedup 1.0000x reference)
import jax
import jax.numpy as jnp
from jax import lax
from jax.experimental import pallas as pl
from jax.experimental.pallas import tpu as pltpu
from jax.experimental.pallas import tpu_sc as plsc

N = 10000
E = 320000
D = 128
DH = D // 2
G = 16
GDIN = 64
N_PAD = 10240
DUMMY = 10000
NC = 2
NS = 16
C = 320
RING = 4
LOOKAHEAD = 2
CHUNKS = RING * (-(-E // (NS * C * RING)))
ROUNDS = CHUNKS // RING
E_PAD = NS * CHUNKS * C
RPT = N_PAD // NS
RB = 1280
GRID = N_PAD // RB
_F32 = jnp.float32
_BF16 = jnp.bfloat16


def _dot(a, b):
    return jnp.dot(a, b, preferred_element_type=_F32)


_SC_MESH = plsc.VectorSubcoreMesh(
    core_axis_name="c", subcore_axis_name="s", num_cores=NC, num_subcores=NS)
_SC_PARAMS = pltpu.CompilerParams(use_tc_tiling_on_sc=False)


def _make_sc_body(with_deg):
    def body(*refs):
        if with_deg:
            (hl, srcr, dstr, zrow, zdeg, onesc, agg_out, deg_out,
             src_v, dst_v, rows_v, ones_v, acc_sh, deg_sh) = refs[:14]
            gsems = refs[14:14 + RING]
            ssems = refs[14 + RING:14 + 2 * RING]
            dsem = refs[14 + 2 * RING]
        else:
            (hl, srcr, dstr, zrow, agg_out,
             src_v, dst_v, rows_v, acc_sh) = refs[:9]
            gsems = refs[9:9 + RING]
            ssems = refs[9 + RING:9 + 2 * RING]
        c = lax.axis_index("c")
        s = lax.axis_index("s")
        pltpu.sync_copy(srcr.at[c, s], src_v)
        pltpu.sync_copy(dstr.at[s], dst_v)
        for b in range(LOOKAHEAD):
            pltpu.async_copy(hl.at[src_v.at[b]], rows_v.at[b], gsems[b])
        if with_deg:
            pltpu.sync_copy(onesc, ones_v)
            pltpu.sync_copy(zdeg.at[pl.ds(s * RPT, RPT)],
                            deg_sh.at[pl.ds(s * RPT, RPT)])
        pltpu.sync_copy(zrow.at[pl.ds(s * RPT, RPT)],
                        acc_sh.at[pl.ds(s * RPT, RPT)])
        plsc.subcore_barrier()

        def round_body(r, carry):
            j0 = r * RING
            for b in range(RING):
                j = j0 + b
                pltpu.make_async_copy(
                    hl.at[src_v.at[j]], rows_v.at[b], gsems[b]).wait()
                pltpu.async_copy(
                    rows_v.at[b], acc_sh.at[dst_v.at[j]], ssems[b], add=True)
                if with_deg:
                    p = b % 2

                    @pl.when(c == p)
                    def _deg(j=j):
                        @pl.when(j >= p + 2)
                        def _wait_prev():
                            pltpu.make_async_copy(
                                ones_v, deg_sh.at[dst_v.at[j]], dsem).wait()

                        pltpu.async_copy(
                            ones_v, deg_sh.at[dst_v.at[j]], dsem, add=True)

                b4 = (b + LOOKAHEAD) % RING

                @pl.when(j + LOOKAHEAD < CHUNKS)
                def _prefetch(j=j, b4=b4):
                    @pl.when(j >= LOOKAHEAD)
                    def _wait_scatter():
                        pltpu.make_async_copy(
                            rows_v.at[b4], acc_sh.at[dst_v.at[j]],
                            ssems[b4]).wait()

                    pltpu.async_copy(hl.at[src_v.at[j + LOOKAHEAD]],
                                     rows_v.at[b4], gsems[b4])

            return carry

        lax.fori_loop(0, ROUNDS, round_body, 0)
        for b in range(RING):
            pltpu.make_async_copy(
                rows_v.at[b], acc_sh.at[dst_v.at[0]], ssems[b]).wait()
        if with_deg:
            pltpu.make_async_copy(ones_v, deg_sh.at[dst_v.at[0]], dsem).wait()
        plsc.subcore_barrier()
        pltpu.sync_copy(acc_sh.at[pl.ds(s * RPT, RPT)],
                        agg_out.at[c, pl.ds(s * RPT, RPT)])
        if with_deg:
            pltpu.sync_copy(deg_sh.at[pl.ds(s * RPT, RPT)],
                            deg_out.at[c, pl.ds(s * RPT, RPT)])

    return body


_sc_agg_deg = pl.kernel(
    _make_sc_body(True),
    out_type=[jax.ShapeDtypeStruct((NC, N_PAD, DH), _BF16),
              jax.ShapeDtypeStruct((NC, N_PAD, 16), _F32)],
    mesh=_SC_MESH,
    scratch_types=[
        pltpu.VMEM((CHUNKS, C), jnp.int32),
        pltpu.VMEM((CHUNKS, C), jnp.int32),
        pltpu.VMEM((RING, C, DH), _BF16),
        pltpu.VMEM((C, 16), _F32),
        pltpu.VMEM_SHARED((N_PAD, DH), _BF16),
        pltpu.VMEM_SHARED((N_PAD, 16), _F32),
    ] + [pltpu.SemaphoreType.DMA] * (2 * RING + 1),
    compiler_params=_SC_PARAMS,
)

_sc_agg = pl.kernel(
    _make_sc_body(False),
    out_type=[jax.ShapeDtypeStruct((NC, N_PAD, DH), _BF16)],
    mesh=_SC_MESH,
    scratch_types=[
        pltpu.VMEM((CHUNKS, C), jnp.int32),
        pltpu.VMEM((CHUNKS, C), jnp.int32),
        pltpu.VMEM((RING, C, DH), _BF16),
        pltpu.VMEM_SHARED((N_PAD, DH), _BF16),
    ] + [pltpu.SemaphoreType.DMA] * (2 * RING),
    compiler_params=_SC_PARAMS,
)


def _tc_pre_body(x_ref, wl_ref, wr_ref, b_ref, hl_ref, pre_ref):
    xv = x_ref[...]
    t = _dot(xv, wl_ref[...]).astype(_BF16)
    hl_ref[0] = t[:, :DH]
    hl_ref[1] = t[:, DH:]
    pre_ref[...] = _dot(xv, wr_ref[...]) + b_ref[...]


def _tc_mid_body(aggp_ref, degp_ref, pre0_ref, wl1_ref, wr1_ref, b1_ref,
                 hl1_ref, pre1_ref, deg_ref):
    agg = jnp.concatenate([aggp_ref[0], aggp_ref[1]], axis=1).astype(_F32)
    deg = jnp.maximum(degp_ref[0] + degp_ref[1], 1.0)
    t = agg / deg[:, 0:1] + pre0_ref[...]
    t = jnp.maximum(t, 0.0)
    ss = jnp.sum(t * t, axis=1, keepdims=True)
    h1 = t / jnp.maximum(jnp.sqrt(ss), 1e-12)
    t1 = _dot(h1, wl1_ref[...]).astype(_BF16)
    hl1_ref[0] = t1[:, :DH]
    hl1_ref[1] = t1[:, DH:]
    pre1_ref[...] = _dot(h1, wr1_ref[...]) + b1_ref[...]
    deg_ref[...] = deg


def _tc_fin_body(aggp_ref, deg_ref, pre1_ref, batch_ref, u_ref, wg_ref,
                 bg_ref, wh_ref, bh_ref, out_ref, gsum_acc, gcnt_acc):
    i = pl.program_id(0)
    agg = jnp.concatenate([aggp_ref[0], aggp_ref[1]], axis=1).astype(_F32)
    t = agg / deg_ref[:, 0:1] + pre1_ref[...]
    ss = jnp.sum(t * t, axis=1, keepdims=True)
    h2 = t / jnp.maximum(jnp.sqrt(ss), 1e-12)
    giota = lax.broadcasted_iota(jnp.int32, (RB, G), 1)
    mask = (batch_ref[...] == giota).astype(_F32)
    part = lax.dot_general(mask, h2, (((0,), (0,)), ((), ())),
                           preferred_element_type=_F32)
    cnt = lax.dot_general(mask, jnp.ones((RB, 1), _F32),
                          (((0,), (0,)), ((), ())),
                          preferred_element_type=_F32)

    @pl.when(i == 0)
    def _init():
        gsum_acc[...] = jnp.zeros_like(gsum_acc)
        gcnt_acc[...] = jnp.zeros_like(gcnt_acc)

    gsum_acc[...] += part
    gcnt_acc[...] += cnt

    @pl.when(i == GRID - 1)
    def _final():
        graph_emb = gsum_acc[...] / jnp.maximum(gcnt_acc[...], 1.0)
        ge = jnp.maximum(_dot(u_ref[...], wg_ref[...]) + bg_ref[...], 0.0)
        fused = jnp.concatenate([graph_emb, ge], axis=1)
        out_ref[...] = _dot(fused, wh_ref[...]) + bh_ref[...]


_row_spec = pl.BlockSpec((RB, D), lambda i: (i, 0))
_half_spec = pl.BlockSpec((NC, RB, DH), lambda i: (0, i, 0))
_deg_spec = pl.BlockSpec((RB, 16), lambda i: (i, 0))
_degp_spec = pl.BlockSpec((NC, RB, 16), lambda i: (0, i, 0))
_w_spec = pl.BlockSpec((D, D), lambda i: (0, 0))
_b_spec = pl.BlockSpec((1, D), lambda i: (0, 0))

_tc_pre = pl.pallas_call(
    _tc_pre_body,
    grid=(GRID,),
    in_specs=[_row_spec, _w_spec, _w_spec, _b_spec],
    out_specs=[_half_spec, _row_spec],
    out_shape=[jax.ShapeDtypeStruct((NC, N_PAD, DH), _BF16),
               jax.ShapeDtypeStruct((N_PAD, D), _F32)],
)

_tc_mid = pl.pallas_call(
    _tc_mid_body,
    grid=(GRID,),
    in_specs=[_half_spec, _degp_spec, _row_spec, _w_spec, _w_spec, _b_spec],
    out_specs=[_half_spec, _row_spec, _deg_spec],
    out_shape=[jax.ShapeDtypeStruct((NC, N_PAD, DH), _BF16),
               jax.ShapeDtypeStruct((N_PAD, D), _F32),
               jax.ShapeDtypeStruct((N_PAD, 16), _F32)],
)

_tc_fin = pl.pallas_call(
    _tc_fin_body,
    grid=(GRID,),
    in_specs=[_half_spec,
              _deg_spec,
              _row_spec,
              pl.BlockSpec((RB, 1), lambda i: (i, 0)),
              pl.BlockSpec((G, GDIN), lambda i: (0, 0)),
              pl.BlockSpec((GDIN, D), lambda i: (0, 0)),
              _b_spec,
              pl.BlockSpec((2 * D, 1), lambda i: (0, 0)),
              pl.BlockSpec((1, 1), lambda i: (0, 0))],
    out_specs=pl.BlockSpec((G, 1), lambda i: (0, 0)),
    out_shape=jax.ShapeDtypeStruct((G, 1), _F32),
    scratch_shapes=[pltpu.VMEM((G, D), _F32), pltpu.VMEM((G, 1), _F32)],
)


def kernel(x, edge_index, u, batch, Wl0, Wr0, b0, Wl1, Wr1, b1, Wg, bg, Wh, bh):
    src = edge_index[0].astype(jnp.int32)
    dst = edge_index[1].astype(jnp.int32)
    pad_e = E_PAD - E
    srcp = jnp.concatenate([src, jnp.zeros((pad_e,), jnp.int32)]).reshape(
        NS, CHUNKS, C)
    srcr = jnp.stack([srcp, srcp + N_PAD])
    dstr = jnp.concatenate([dst, jnp.full((pad_e,), DUMMY, jnp.int32)]).reshape(
        NS, CHUNKS, C)
    xpad = jnp.zeros((N_PAD, D), _F32).at[:N].set(x)
    batch_p = jnp.full((N_PAD, 1), G, jnp.int32).at[:N, 0].set(
        batch.astype(jnp.int32))
    zrow = jnp.zeros((N_PAD, DH), _BF16)
    zdeg = jnp.zeros((N_PAD, 16), _F32)
    onesc = jnp.ones((C, 16), _F32)

    hl0, pre0 = _tc_pre(xpad, Wl0, Wr0, b0.reshape(1, D))
    agg0, deg0 = _sc_agg_deg(hl0.reshape(NC * N_PAD, DH), srcr, dstr,
                             zrow, zdeg, onesc)
    hl1, pre1, deg = _tc_mid(agg0, deg0, pre0, Wl1, Wr1, b1.reshape(1, D))
    (agg1,) = _sc_agg(hl1.reshape(NC * N_PAD, DH), srcr, dstr, zrow)
    out = _tc_fin(agg1, deg, pre1, batch_p, u, Wg, bg.reshape(1, D),
                  Wh, bh.reshape(1, 1))
    return out


# speedup vs baseline: 1.2167x; 1.2167x over previous
import jax
import jax.numpy as jnp
from jax import lax
from jax.experimental import pallas as pl
from jax.experimental.pallas import tpu as pltpu
from jax.experimental.pallas import tpu_sc as plsc

N = 10000
E = 320000
D = 128
DH = D // 2
G = 16
GDIN = 64
N_PAD = 10240
DUMMY = 10000
NC = 2
NS = 16
C = 256
RING = 4
LOOKAHEAD = 2
CHUNKS = RING * (-(-E // (NS * C * RING)))
ROUNDS = CHUNKS // RING
E_PAD = NS * CHUNKS * C
RPT = N_PAD // NS
RB = 2560
GRID = N_PAD // RB
_F32 = jnp.float32
_BF16 = jnp.bfloat16


def _dot(a, b):
    return jnp.dot(a, b, preferred_element_type=_F32)


_SC_MESH = plsc.VectorSubcoreMesh(
    core_axis_name="c", subcore_axis_name="s", num_cores=NC, num_subcores=NS)
_SC_PARAMS = pltpu.CompilerParams(use_tc_tiling_on_sc=False)


def _make_sc_body(with_deg):
    def body(*refs):
        if with_deg:
            (hl, srcr, dstr, zrow, zdeg, onesc, agg_out, deg_out,
             src_v, dst_v, rows_v, ones_v, acc_sh, deg_sh) = refs[:14]
            gsems = refs[14:14 + RING]
            ssems = refs[14 + RING:14 + 2 * RING]
            dsem = refs[14 + 2 * RING]
        else:
            (hl, srcr, dstr, zrow, agg_out,
             src_v, dst_v, rows_v, acc_sh) = refs[:9]
            gsems = refs[9:9 + RING]
            ssems = refs[9 + RING:9 + 2 * RING]
        c = lax.axis_index("c")
        s = lax.axis_index("s")
        pltpu.sync_copy(srcr.at[c, s], src_v)
        pltpu.sync_copy(dstr.at[s], dst_v)
        for b in range(LOOKAHEAD):
            pltpu.async_copy(hl.at[src_v.at[b]], rows_v.at[b], gsems[b])
        if with_deg:
            pltpu.sync_copy(onesc, ones_v)
            pltpu.sync_copy(zdeg.at[pl.ds(s * RPT, RPT)],
                            deg_sh.at[pl.ds(s * RPT, RPT)])
        pltpu.sync_copy(zrow.at[pl.ds(s * RPT, RPT)],
                        acc_sh.at[pl.ds(s * RPT, RPT)])
        plsc.subcore_barrier()

        def round_body(r, carry):
            j0 = r * RING
            for b in range(RING):
                j = j0 + b
                pltpu.make_async_copy(
                    hl.at[src_v.at[j]], rows_v.at[b], gsems[b]).wait()
                pltpu.async_copy(
                    rows_v.at[b], acc_sh.at[dst_v.at[j]], ssems[b], add=True)
                if with_deg:
                    p = b % 2

                    @pl.when(c == p)
                    def _deg(j=j):
                        @pl.when(j >= p + 2)
                        def _wait_prev():
                            pltpu.make_async_copy(
                                ones_v, deg_sh.at[dst_v.at[j]], dsem).wait()

                        pltpu.async_copy(
                            ones_v, deg_sh.at[dst_v.at[j]], dsem, add=True)

                b4 = (b + LOOKAHEAD) % RING

                @pl.when(j + LOOKAHEAD < CHUNKS)
                def _prefetch(j=j, b4=b4):
                    @pl.when(j >= LOOKAHEAD)
                    def _wait_scatter():
                        pltpu.make_async_copy(
                            rows_v.at[b4], acc_sh.at[dst_v.at[j]],
                            ssems[b4]).wait()

                    pltpu.async_copy(hl.at[src_v.at[j + LOOKAHEAD]],
                                     rows_v.at[b4], gsems[b4])

            return carry

        lax.fori_loop(0, ROUNDS, round_body, 0)
        for b in range(RING):
            pltpu.make_async_copy(
                rows_v.at[b], acc_sh.at[dst_v.at[0]], ssems[b]).wait()
        if with_deg:
            pltpu.make_async_copy(ones_v, deg_sh.at[dst_v.at[0]], dsem).wait()
        plsc.subcore_barrier()
        pltpu.sync_copy(acc_sh.at[pl.ds(s * RPT, RPT)],
                        agg_out.at[c, pl.ds(s * RPT, RPT)])
        if with_deg:
            pltpu.sync_copy(deg_sh.at[pl.ds(s * RPT, RPT)],
                            deg_out.at[c, pl.ds(s * RPT, RPT)])

    return body


_sc_agg_deg = pl.kernel(
    _make_sc_body(True),
    out_type=[jax.ShapeDtypeStruct((NC, N_PAD, DH), _BF16),
              jax.ShapeDtypeStruct((NC, N_PAD, 16), _F32)],
    mesh=_SC_MESH,
    scratch_types=[
        pltpu.VMEM((CHUNKS, C), jnp.int32),
        pltpu.VMEM((CHUNKS, C), jnp.int32),
        pltpu.VMEM((RING, C, DH), _BF16),
        pltpu.VMEM((C, 16), _F32),
        pltpu.VMEM_SHARED((N_PAD, DH), _BF16),
        pltpu.VMEM_SHARED((N_PAD, 16), _F32),
    ] + [pltpu.SemaphoreType.DMA] * (2 * RING + 1),
    compiler_params=_SC_PARAMS,
)

_sc_agg = pl.kernel(
    _make_sc_body(False),
    out_type=[jax.ShapeDtypeStruct((NC, N_PAD, DH), _BF16)],
    mesh=_SC_MESH,
    scratch_types=[
        pltpu.VMEM((CHUNKS, C), jnp.int32),
        pltpu.VMEM((CHUNKS, C), jnp.int32),
        pltpu.VMEM((RING, C, DH), _BF16),
        pltpu.VMEM_SHARED((N_PAD, DH), _BF16),
    ] + [pltpu.SemaphoreType.DMA] * (2 * RING),
    compiler_params=_SC_PARAMS,
)


def _tc_pre_body(x_ref, wl_ref, wr_ref, b_ref, hl_ref, pre_ref):
    xv = x_ref[...]
    t = _dot(xv, wl_ref[...]).astype(_BF16)
    hl_ref[0] = t[:, :DH]
    hl_ref[1] = t[:, DH:]
    pre_ref[...] = _dot(xv, wr_ref[...]) + b_ref[...]


def _tc_mid_body(aggp_ref, degp_ref, pre0_ref, wl1_ref, wr1_ref, b1_ref,
                 hl1_ref, pre1_ref, deg_ref):
    agg = jnp.concatenate([aggp_ref[0], aggp_ref[1]], axis=1).astype(_F32)
    deg = jnp.maximum(degp_ref[0] + degp_ref[1], 1.0)
    t = agg / deg[:, 0:1] + pre0_ref[...]
    t = jnp.maximum(t, 0.0)
    ss = jnp.sum(t * t, axis=1, keepdims=True)
    h1 = t / jnp.maximum(jnp.sqrt(ss), 1e-12)
    t1 = _dot(h1, wl1_ref[...]).astype(_BF16)
    hl1_ref[0] = t1[:, :DH]
    hl1_ref[1] = t1[:, DH:]
    pre1_ref[...] = _dot(h1, wr1_ref[...]) + b1_ref[...]
    deg_ref[...] = deg


def _tc_fin_body(aggp_ref, deg_ref, pre1_ref, batch_ref, u_ref, wg_ref,
                 bg_ref, wh_ref, bh_ref, out_ref, gsum_acc, gcnt_acc):
    i = pl.program_id(0)
    agg = jnp.concatenate([aggp_ref[0], aggp_ref[1]], axis=1).astype(_F32)
    t = agg / deg_ref[:, 0:1] + pre1_ref[...]
    ss = jnp.sum(t * t, axis=1, keepdims=True)
    h2 = t / jnp.maximum(jnp.sqrt(ss), 1e-12)
    giota = lax.broadcasted_iota(jnp.int32, (RB, G), 1)
    mask = (batch_ref[...] == giota).astype(_F32)
    part = lax.dot_general(mask, h2, (((0,), (0,)), ((), ())),
                           preferred_element_type=_F32)
    cnt = lax.dot_general(mask, jnp.ones((RB, 1), _F32),
                          (((0,), (0,)), ((), ())),
                          preferred_element_type=_F32)

    @pl.when(i == 0)
    def _init():
        gsum_acc[...] = jnp.zeros_like(gsum_acc)
        gcnt_acc[...] = jnp.zeros_like(gcnt_acc)

    gsum_acc[...] += part
    gcnt_acc[...] += cnt

    @pl.when(i == GRID - 1)
    def _final():
        graph_emb = gsum_acc[...] / jnp.maximum(gcnt_acc[...], 1.0)
        ge = jnp.maximum(_dot(u_ref[...], wg_ref[...]) + bg_ref[...], 0.0)
        fused = jnp.concatenate([graph_emb, ge], axis=1)
        out_ref[...] = _dot(fused, wh_ref[...]) + bh_ref[...]


_row_spec = pl.BlockSpec((RB, D), lambda i: (i, 0))
_half_spec = pl.BlockSpec((NC, RB, DH), lambda i: (0, i, 0))
_deg_spec = pl.BlockSpec((RB, 16), lambda i: (i, 0))
_degp_spec = pl.BlockSpec((NC, RB, 16), lambda i: (0, i, 0))
_w_spec = pl.BlockSpec((D, D), lambda i: (0, 0))
_b_spec = pl.BlockSpec((1, D), lambda i: (0, 0))

_tc_pre = pl.pallas_call(
    _tc_pre_body,
    grid=(GRID,),
    in_specs=[_row_spec, _w_spec, _w_spec, _b_spec],
    out_specs=[_half_spec, _row_spec],
    out_shape=[jax.ShapeDtypeStruct((NC, N_PAD, DH), _BF16),
               jax.ShapeDtypeStruct((N_PAD, D), _F32)],
)

_tc_mid = pl.pallas_call(
    _tc_mid_body,
    grid=(GRID,),
    in_specs=[_half_spec, _degp_spec, _row_spec, _w_spec, _w_spec, _b_spec],
    out_specs=[_half_spec, _row_spec, _deg_spec],
    out_shape=[jax.ShapeDtypeStruct((NC, N_PAD, DH), _BF16),
               jax.ShapeDtypeStruct((N_PAD, D), _F32),
               jax.ShapeDtypeStruct((N_PAD, 16), _F32)],
)

_tc_fin = pl.pallas_call(
    _tc_fin_body,
    grid=(GRID,),
    in_specs=[_half_spec,
              _deg_spec,
              _row_spec,
              pl.BlockSpec((RB, 1), lambda i: (i, 0)),
              pl.BlockSpec((G, GDIN), lambda i: (0, 0)),
              pl.BlockSpec((GDIN, D), lambda i: (0, 0)),
              _b_spec,
              pl.BlockSpec((2 * D, 1), lambda i: (0, 0)),
              pl.BlockSpec((1, 1), lambda i: (0, 0))],
    out_specs=pl.BlockSpec((G, 1), lambda i: (0, 0)),
    out_shape=jax.ShapeDtypeStruct((G, 1), _F32),
    scratch_shapes=[pltpu.VMEM((G, D), _F32), pltpu.VMEM((G, 1), _F32)],
)


def kernel(x, edge_index, u, batch, Wl0, Wr0, b0, Wl1, Wr1, b1, Wg, bg, Wh, bh):
    src = edge_index[0].astype(jnp.int32)
    dst = edge_index[1].astype(jnp.int32)
    pad_e = E_PAD - E
    srcp = jnp.concatenate([src, jnp.zeros((pad_e,), jnp.int32)]).reshape(
        NS, CHUNKS, C)
    srcr = jnp.stack([srcp, srcp + N_PAD])
    dstr = jnp.concatenate([dst, jnp.full((pad_e,), DUMMY, jnp.int32)]).reshape(
        NS, CHUNKS, C)
    xpad = jnp.zeros((N_PAD, D), _F32).at[:N].set(x)
    batch_p = jnp.full((N_PAD, 1), G, jnp.int32).at[:N, 0].set(
        batch.astype(jnp.int32))
    zrow = jnp.zeros((N_PAD, DH), _BF16)
    zdeg = jnp.zeros((N_PAD, 16), _F32)
    onesc = jnp.ones((C, 16), _F32)

    hl0, pre0 = _tc_pre(xpad, Wl0, Wr0, b0.reshape(1, D))
    agg0, deg0 = _sc_agg_deg(hl0.reshape(NC * N_PAD, DH), srcr, dstr,
                             zrow, zdeg, onesc)
    hl1, pre1, deg = _tc_mid(agg0, deg0, pre0, Wl1, Wr1, b1.reshape(1, D))
    (agg1,) = _sc_agg(hl1.reshape(NC * N_PAD, DH), srcr, dstr, zrow)
    out = _tc_fin(agg1, deg, pre1, batch_p, u, Wg, bg.reshape(1, D),
                  Wh, bh.reshape(1, 1))
    return out


# speedup vs baseline: 1.2187x; 1.0017x over previous
import jax
import jax.numpy as jnp
from jax import lax
from jax.experimental import pallas as pl
from jax.experimental.pallas import tpu as pltpu
from jax.experimental.pallas import tpu_sc as plsc

N = 10000
E = 320000
D = 128
DH = D // 2
G = 16
GDIN = 64
N_PAD = 10240
DUMMY = 10000
NC = 2
NS = 16
C = 256
RING = 4
LOOKAHEAD = 2
CHUNKS = RING * (-(-E // (NS * C * RING)))
ROUNDS = CHUNKS // RING
E_PAD = NS * CHUNKS * C
RPT = N_PAD // NS
RB = 5120
GRID = N_PAD // RB
_F32 = jnp.float32
_BF16 = jnp.bfloat16


def _dot(a, b):
    return jnp.dot(a, b, preferred_element_type=_F32)


_SC_MESH = plsc.VectorSubcoreMesh(
    core_axis_name="c", subcore_axis_name="s", num_cores=NC, num_subcores=NS)
_SC_PARAMS = pltpu.CompilerParams(use_tc_tiling_on_sc=False)


def _make_sc_body(with_deg):
    def body(*refs):
        if with_deg:
            (hl, srcr, dstr, zrow, zdeg, onesc, agg_out, deg_out,
             src_v, dst_v, rows_v, ones_v, acc_sh, deg_sh) = refs[:14]
            gsems = refs[14:14 + RING]
            ssems = refs[14 + RING:14 + 2 * RING]
            dsem = refs[14 + 2 * RING]
        else:
            (hl, srcr, dstr, zrow, agg_out,
             src_v, dst_v, rows_v, acc_sh) = refs[:9]
            gsems = refs[9:9 + RING]
            ssems = refs[9 + RING:9 + 2 * RING]
        c = lax.axis_index("c")
        s = lax.axis_index("s")
        pltpu.sync_copy(srcr.at[c, s], src_v)
        pltpu.sync_copy(dstr.at[s], dst_v)
        for b in range(LOOKAHEAD):
            pltpu.async_copy(hl.at[src_v.at[b]], rows_v.at[b], gsems[b])
        if with_deg:
            pltpu.sync_copy(onesc, ones_v)
            pltpu.sync_copy(zdeg.at[pl.ds(s * RPT, RPT)],
                            deg_sh.at[pl.ds(s * RPT, RPT)])
        pltpu.sync_copy(zrow.at[pl.ds(s * RPT, RPT)],
                        acc_sh.at[pl.ds(s * RPT, RPT)])
        plsc.subcore_barrier()

        def round_body(r, carry):
            j0 = r * RING
            for b in range(RING):
                j = j0 + b
                pltpu.make_async_copy(
                    hl.at[src_v.at[j]], rows_v.at[b], gsems[b]).wait()
                pltpu.async_copy(
                    rows_v.at[b], acc_sh.at[dst_v.at[j]], ssems[b], add=True)
                if with_deg:
                    p = b % 2

                    @pl.when(c == p)
                    def _deg(j=j):
                        @pl.when(j >= p + 2)
                        def _wait_prev():
                            pltpu.make_async_copy(
                                ones_v, deg_sh.at[dst_v.at[j]], dsem).wait()

                        pltpu.async_copy(
                            ones_v, deg_sh.at[dst_v.at[j]], dsem, add=True)

                b4 = (b + LOOKAHEAD) % RING

                @pl.when(j + LOOKAHEAD < CHUNKS)
                def _prefetch(j=j, b4=b4):
                    @pl.when(j >= LOOKAHEAD)
                    def _wait_scatter():
                        pltpu.make_async_copy(
                            rows_v.at[b4], acc_sh.at[dst_v.at[j]],
                            ssems[b4]).wait()

                    pltpu.async_copy(hl.at[src_v.at[j + LOOKAHEAD]],
                                     rows_v.at[b4], gsems[b4])

            return carry

        lax.fori_loop(0, ROUNDS, round_body, 0)
        for b in range(RING):
            pltpu.make_async_copy(
                rows_v.at[b], acc_sh.at[dst_v.at[0]], ssems[b]).wait()
        if with_deg:
            pltpu.make_async_copy(ones_v, deg_sh.at[dst_v.at[0]], dsem).wait()
        plsc.subcore_barrier()
        pltpu.sync_copy(acc_sh.at[pl.ds(s * RPT, RPT)],
                        agg_out.at[c, pl.ds(s * RPT, RPT)])
        if with_deg:
            pltpu.sync_copy(deg_sh.at[pl.ds(s * RPT, RPT)],
                            deg_out.at[c, pl.ds(s * RPT, RPT)])

    return body


_sc_agg_deg = pl.kernel(
    _make_sc_body(True),
    out_type=[jax.ShapeDtypeStruct((NC, N_PAD, DH), _BF16),
              jax.ShapeDtypeStruct((NC, N_PAD, 16), _F32)],
    mesh=_SC_MESH,
    scratch_types=[
        pltpu.VMEM((CHUNKS, C), jnp.int32),
        pltpu.VMEM((CHUNKS, C), jnp.int32),
        pltpu.VMEM((RING, C, DH), _BF16),
        pltpu.VMEM((C, 16), _F32),
        pltpu.VMEM_SHARED((N_PAD, DH), _BF16),
        pltpu.VMEM_SHARED((N_PAD, 16), _F32),
    ] + [pltpu.SemaphoreType.DMA] * (2 * RING + 1),
    compiler_params=_SC_PARAMS,
)

_sc_agg = pl.kernel(
    _make_sc_body(False),
    out_type=[jax.ShapeDtypeStruct((NC, N_PAD, DH), _BF16)],
    mesh=_SC_MESH,
    scratch_types=[
        pltpu.VMEM((CHUNKS, C), jnp.int32),
        pltpu.VMEM((CHUNKS, C), jnp.int32),
        pltpu.VMEM((RING, C, DH), _BF16),
        pltpu.VMEM_SHARED((N_PAD, DH), _BF16),
    ] + [pltpu.SemaphoreType.DMA] * (2 * RING),
    compiler_params=_SC_PARAMS,
)


def _tc_pre_body(x_ref, wl_ref, wr_ref, b_ref, hl_ref, pre_ref):
    xv = x_ref[...]
    t = _dot(xv, wl_ref[...]).astype(_BF16)
    hl_ref[0] = t[:, :DH]
    hl_ref[1] = t[:, DH:]
    pre_ref[...] = _dot(xv, wr_ref[...]) + b_ref[...]


def _tc_mid_body(aggp_ref, degp_ref, pre0_ref, wl1_ref, wr1_ref, b1_ref,
                 hl1_ref, pre1_ref, deg_ref):
    agg = jnp.concatenate([aggp_ref[0], aggp_ref[1]], axis=1).astype(_F32)
    deg = jnp.maximum(degp_ref[0] + degp_ref[1], 1.0)
    t = agg / deg[:, 0:1] + pre0_ref[...]
    t = jnp.maximum(t, 0.0)
    ss = jnp.sum(t * t, axis=1, keepdims=True)
    h1 = t / jnp.maximum(jnp.sqrt(ss), 1e-12)
    t1 = _dot(h1, wl1_ref[...]).astype(_BF16)
    hl1_ref[0] = t1[:, :DH]
    hl1_ref[1] = t1[:, DH:]
    pre1_ref[...] = _dot(h1, wr1_ref[...]) + b1_ref[...]
    deg_ref[...] = deg


def _tc_fin_body(aggp_ref, deg_ref, pre1_ref, batch_ref, u_ref, wg_ref,
                 bg_ref, wh_ref, bh_ref, out_ref, gsum_acc, gcnt_acc):
    i = pl.program_id(0)
    agg = jnp.concatenate([aggp_ref[0], aggp_ref[1]], axis=1).astype(_F32)
    t = agg / deg_ref[:, 0:1] + pre1_ref[...]
    ss = jnp.sum(t * t, axis=1, keepdims=True)
    h2 = t / jnp.maximum(jnp.sqrt(ss), 1e-12)
    giota = lax.broadcasted_iota(jnp.int32, (RB, G), 1)
    mask = (batch_ref[...] == giota).astype(_F32)
    part = lax.dot_general(mask, h2, (((0,), (0,)), ((), ())),
                           preferred_element_type=_F32)
    cnt = lax.dot_general(mask, jnp.ones((RB, 1), _F32),
                          (((0,), (0,)), ((), ())),
                          preferred_element_type=_F32)

    @pl.when(i == 0)
    def _init():
        gsum_acc[...] = jnp.zeros_like(gsum_acc)
        gcnt_acc[...] = jnp.zeros_like(gcnt_acc)

    gsum_acc[...] += part
    gcnt_acc[...] += cnt

    @pl.when(i == GRID - 1)
    def _final():
        graph_emb = gsum_acc[...] / jnp.maximum(gcnt_acc[...], 1.0)
        ge = jnp.maximum(_dot(u_ref[...], wg_ref[...]) + bg_ref[...], 0.0)
        fused = jnp.concatenate([graph_emb, ge], axis=1)
        out_ref[...] = _dot(fused, wh_ref[...]) + bh_ref[...]


_row_spec = pl.BlockSpec((RB, D), lambda i: (i, 0))
_half_spec = pl.BlockSpec((NC, RB, DH), lambda i: (0, i, 0))
_deg_spec = pl.BlockSpec((RB, 16), lambda i: (i, 0))
_degp_spec = pl.BlockSpec((NC, RB, 16), lambda i: (0, i, 0))
_w_spec = pl.BlockSpec((D, D), lambda i: (0, 0))
_b_spec = pl.BlockSpec((1, D), lambda i: (0, 0))

_tc_pre = pl.pallas_call(
    _tc_pre_body,
    grid=(GRID,),
    in_specs=[_row_spec, _w_spec, _w_spec, _b_spec],
    out_specs=[_half_spec, _row_spec],
    out_shape=[jax.ShapeDtypeStruct((NC, N_PAD, DH), _BF16),
               jax.ShapeDtypeStruct((N_PAD, D), _F32)],
)

_tc_mid = pl.pallas_call(
    _tc_mid_body,
    grid=(GRID,),
    in_specs=[_half_spec, _degp_spec, _row_spec, _w_spec, _w_spec, _b_spec],
    out_specs=[_half_spec, _row_spec, _deg_spec],
    out_shape=[jax.ShapeDtypeStruct((NC, N_PAD, DH), _BF16),
               jax.ShapeDtypeStruct((N_PAD, D), _F32),
               jax.ShapeDtypeStruct((N_PAD, 16), _F32)],
)

_tc_fin = pl.pallas_call(
    _tc_fin_body,
    grid=(GRID,),
    in_specs=[_half_spec,
              _deg_spec,
              _row_spec,
              pl.BlockSpec((RB, 1), lambda i: (i, 0)),
              pl.BlockSpec((G, GDIN), lambda i: (0, 0)),
              pl.BlockSpec((GDIN, D), lambda i: (0, 0)),
              _b_spec,
              pl.BlockSpec((2 * D, 1), lambda i: (0, 0)),
              pl.BlockSpec((1, 1), lambda i: (0, 0))],
    out_specs=pl.BlockSpec((G, 1), lambda i: (0, 0)),
    out_shape=jax.ShapeDtypeStruct((G, 1), _F32),
    scratch_shapes=[pltpu.VMEM((G, D), _F32), pltpu.VMEM((G, 1), _F32)],
)


def kernel(x, edge_index, u, batch, Wl0, Wr0, b0, Wl1, Wr1, b1, Wg, bg, Wh, bh):
    src = edge_index[0].astype(jnp.int32)
    dst = edge_index[1].astype(jnp.int32)
    pad_e = E_PAD - E
    srcp = jnp.concatenate([src, jnp.zeros((pad_e,), jnp.int32)]).reshape(
        NS, CHUNKS, C)
    srcr = jnp.stack([srcp, srcp + N_PAD])
    dstr = jnp.concatenate([dst, jnp.full((pad_e,), DUMMY, jnp.int32)]).reshape(
        NS, CHUNKS, C)
    xpad = jnp.zeros((N_PAD, D), _F32).at[:N].set(x)
    batch_p = jnp.full((N_PAD, 1), G, jnp.int32).at[:N, 0].set(
        batch.astype(jnp.int32))
    zrow = jnp.zeros((N_PAD, DH), _BF16)
    zdeg = jnp.zeros((N_PAD, 16), _F32)
    onesc = jnp.ones((C, 16), _F32)

    hl0, pre0 = _tc_pre(xpad, Wl0, Wr0, b0.reshape(1, D))
    agg0, deg0 = _sc_agg_deg(hl0.reshape(NC * N_PAD, DH), srcr, dstr,
                             zrow, zdeg, onesc)
    hl1, pre1, deg = _tc_mid(agg0, deg0, pre0, Wl1, Wr1, b1.reshape(1, D))
    (agg1,) = _sc_agg(hl1.reshape(NC * N_PAD, DH), srcr, dstr, zrow)
    out = _tc_fin(agg1, deg, pre1, batch_p, u, Wg, bg.reshape(1, D),
                  Wh, bh.reshape(1, 1))
    return out
